# Initial kernel scaffold; baseline (speedup 1.0000x reference)
import jax
import jax.numpy as jnp
from jax.experimental import pallas as pl


def kernel(node_feat, edge_attr, W_ea1, b_ea1, W_ea2, b_ea2, W_src, b_src, W_dst, b_dst, W_te1, b_te1, W_te2, b_te2, W_phid, b_phid, W_phie, b_phie, W_phi, b_phi, edge_index):
    raise NotImplementedError("write your pallas kernel here")


# speedup vs baseline: 2.7387x; 2.7387x over previous
import functools

import jax
import jax.numpy as jnp
from jax import lax
from jax.experimental import pallas as pl
from jax.experimental.pallas import tpu as pltpu
from jax.experimental.pallas import tpu_sc as plsc

F32 = jnp.float32


def _node_proj(x, W_src, b_src, W_dst, b_dst):
    N, D = x.shape
    H = W_src.shape[1]

    def body(x_ref, ws_ref, bs_ref, wd_ref, bd_ref, os_ref, od_ref):
        xv = x_ref[...]
        os_ref[...] = jnp.dot(xv, ws_ref[...], preferred_element_type=F32) + bs_ref[...]
        od_ref[...] = jnp.dot(xv, wd_ref[...], preferred_element_type=F32) + bd_ref[...]

    return pl.pallas_call(
        body,
        out_shape=(jax.ShapeDtypeStruct((N, H), F32),
                   jax.ShapeDtypeStruct((N, H), F32)),
    )(x, W_src, b_src.reshape(1, -1), W_dst, b_dst.reshape(1, -1))


def _edge_mlp(edge_attr, s, W_ea1, b_ea1, W_ea2, b_ea2, W_te1, b_te1, W_te2, b_te2):
    E, DE = edge_attr.shape
    H = W_ea1.shape[1]
    D = W_te2.shape[1]
    K = 2560
    assert E % K == 0
    grid = E // K

    def body(ea_ref, s_ref, w1, c1, w2, c2, w3, c3, w4, c4, out_ref):
        ec = jnp.dot(jax.nn.relu(jnp.dot(ea_ref[...], w1[...], preferred_element_type=F32) + c1[...]),
                     w2[...], preferred_element_type=F32) + c2[...]
        t = jax.nn.relu(ec + s_ref[...])
        u = jax.nn.relu(jnp.dot(t, w3[...], preferred_element_type=F32) + c3[...])
        out_ref[...] = jnp.dot(u, w4[...], preferred_element_type=F32) + c4[...]

    wspec = lambda r, c: pl.BlockSpec((r, c), lambda i: (0, 0))
    return pl.pallas_call(
        body,
        grid=(grid,),
        in_specs=[
            pl.BlockSpec((K, DE), lambda i: (i, 0)),
            pl.BlockSpec((K, H), lambda i: (i, 0)),
            wspec(DE, H), wspec(1, H), wspec(H, H), wspec(1, H),
            wspec(H, H), wspec(1, H), wspec(H, D), wspec(1, D),
        ],
        out_specs=pl.BlockSpec((K, D), lambda i: (i, 0)),
        out_shape=jax.ShapeDtypeStruct((E, D), F32),
    )(edge_attr, s,
      W_ea1, b_ea1.reshape(1, -1), W_ea2, b_ea2.reshape(1, -1),
      W_te1, b_te1.reshape(1, -1), W_te2, b_te2.reshape(1, -1))


def _final_mlp(x, agg_parts, W_phid, b_phid, W_phie, b_phie, W_phi, b_phi):
    N, D = x.shape
    H = W_phid.shape[1]

    def body(x_ref, a_ref, wd, bd, we, be, wp, bp, out_ref):
        agg = a_ref[0] + a_ref[1]
        h = jax.nn.relu(jnp.dot(x_ref[...], wd[...], preferred_element_type=F32)
                        + jnp.dot(agg, we[...], preferred_element_type=F32)
                        + bd[...] + be[...])
        out_ref[...] = jnp.dot(h, wp[...], preferred_element_type=F32) + bp[...]

    return pl.pallas_call(
        body,
        out_shape=jax.ShapeDtypeStruct((N, D), F32),
    )(x, agg_parts, W_phid, b_phid.reshape(1, -1),
      W_phie, b_phie.reshape(1, -1), W_phi, b_phi.reshape(1, -1))


_C = 128


def _sc_gather_sum(xw_src, xw_dst, src, dst, NC, NS):
    N, H = xw_src.shape
    E = src.shape[0]
    assert E % _C == 0
    TOT = E // _C
    NW = NC * NS
    KMAX = (TOT + NW - 1) // NW
    mesh = plsc.VectorSubcoreMesh(core_axis_name="c", subcore_axis_name="s")

    @functools.partial(
        pl.kernel, mesh=mesh,
        out_type=jax.ShapeDtypeStruct((E, H), F32),
        scratch_types=[
            pltpu.VMEM((_C,), jnp.int32), pltpu.VMEM((_C,), jnp.int32),
            pltpu.VMEM((_C, H), F32), pltpu.VMEM((_C, H), F32),
            pltpu.SemaphoreType.DMA,
        ],
    )
    def k(xs_hbm, xd_hbm, src_hbm, dst_hbm, out_hbm, sidx, didx, gs, gd, sem):
        wid = lax.axis_index("s") * NC + lax.axis_index("c")

        def step(kk, carry):
            chunk = kk * NW + wid

            @pl.when(chunk < TOT)
            def _():
                base = chunk * _C
                pltpu.sync_copy(src_hbm.at[pl.ds(base, _C)], sidx)
                pltpu.sync_copy(dst_hbm.at[pl.ds(base, _C)], didx)
                cp1 = pltpu.async_copy(xs_hbm.at[sidx], gs, sem)
                cp2 = pltpu.async_copy(xd_hbm.at[didx], gd, sem)
                cp1.wait()
                cp2.wait()

                def row(r, c):
                    for j in range(H // 16):
                        sl = pl.ds(j * 16, 16)
                        gs[r, sl] = gs[r, sl] + gd[r, sl]
                    return c

                lax.fori_loop(0, _C, row, 0)
                pltpu.sync_copy(gs, out_hbm.at[pl.ds(base, _C)])

            return carry

        lax.fori_loop(0, KMAX, step, 0)

    return k(xw_src, xw_dst, src, dst)


def _sc_scatter_agg(x, e_emb, src, dst, NC, NS):
    N, H = x.shape
    E = src.shape[0]
    assert E % _C == 0
    TOT = E // _C
    per_core = TOT // NC
    KMAX = (per_core + NS - 1) // NS
    RPT = ((N + NS * _C - 1) // (NS * _C)) * _C
    NPAD = NS * RPT
    mesh = plsc.VectorSubcoreMesh(core_axis_name="c", subcore_axis_name="s")

    @functools.partial(
        pl.kernel, mesh=mesh,
        out_type=jax.ShapeDtypeStruct((NC, NPAD, H), F32),
        scratch_types=[
            pltpu.VMEM((_C,), jnp.int32), pltpu.VMEM((_C,), jnp.int32),
            pltpu.VMEM((_C, H), F32), pltpu.VMEM((_C, H), F32),
            pltpu.VMEM_SHARED((NPAD, H), F32),
            pltpu.SemaphoreType.DMA,
        ],
    )
    def k(x_hbm, ee_hbm, src_hbm, dst_hbm, out_hbm, sidx, didx, gx, em, acc, sem):
        cid = lax.axis_index("c")
        sid = lax.axis_index("s")

        def zrow(r, c):
            for j in range(H // 16):
                em[r, pl.ds(j * 16, 16)] = jnp.zeros((16,), F32)
            return c

        lax.fori_loop(0, _C, zrow, 0)
        for j in range(RPT // _C):
            pltpu.sync_copy(em, acc.at[pl.ds(sid * RPT + j * _C, _C)])
        plsc.subcore_barrier()

        def step(kk, carry):
            chunk = cid * per_core + kk * NS + sid

            @pl.when(chunk < (cid + 1) * per_core)
            def _():
                base = chunk * _C
                pltpu.sync_copy(src_hbm.at[pl.ds(base, _C)], sidx)
                pltpu.sync_copy(dst_hbm.at[pl.ds(base, _C)], didx)
                cp = pltpu.async_copy(x_hbm.at[sidx], gx, sem)
                pltpu.sync_copy(ee_hbm.at[pl.ds(base, _C)], em)
                cp.wait()

                def row(r, c):
                    for j in range(H // 16):
                        sl = pl.ds(j * 16, 16)
                        em[r, sl] = em[r, sl] * gx[r, sl]
                    return c

                lax.fori_loop(0, _C, row, 0)
                pltpu.sync_copy(em, acc.at[didx], add=True)

            return carry

        lax.fori_loop(0, KMAX, step, 0)
        plsc.subcore_barrier()
        for j in range(RPT // _C):
            off = sid * RPT + j * _C
            pltpu.sync_copy(acc.at[pl.ds(off, _C)],
                            out_hbm.at[cid, pl.ds(off, _C)])

    return k(x, e_emb, src, dst)[:, :N, :]


def kernel(node_feat, edge_attr, W_ea1, b_ea1, W_ea2, b_ea2, W_src, b_src,
           W_dst, b_dst, W_te1, b_te1, W_te2, b_te2, W_phid, b_phid,
           W_phie, b_phie, W_phi, b_phi, edge_index):
    info = plsc.get_sparse_core_info()
    NC, NS = info.num_cores, info.num_subcores
    src = edge_index[0]
    dst = edge_index[1]
    xw_src, xw_dst = _node_proj(node_feat, W_src, b_src, W_dst, b_dst)
    s = _sc_gather_sum(xw_src, xw_dst, src, dst, NC, NS)
    e_emb = _edge_mlp(edge_attr, s, W_ea1, b_ea1, W_ea2, b_ea2,
                      W_te1, b_te1, W_te2, b_te2)
    agg_parts = _sc_scatter_agg(node_feat, e_emb, src, dst, NC, NS)
    return _final_mlp(node_feat, agg_parts, W_phid, b_phid,
                      W_phie, b_phie, W_phi, b_phi)


# speedup vs baseline: 3.4289x; 1.2520x over previous
import functools

import jax
import jax.numpy as jnp
from jax import lax
from jax.experimental import pallas as pl
from jax.experimental.pallas import tpu as pltpu
from jax.experimental.pallas import tpu_sc as plsc

F32 = jnp.float32


def _node_proj(x, W_src, b_src, W_dst, b_dst):
    N, D = x.shape
    H = W_src.shape[1]

    def body(x_ref, ws_ref, bs_ref, wd_ref, bd_ref, os_ref, od_ref):
        xv = x_ref[...]
        os_ref[...] = jnp.dot(xv, ws_ref[...], preferred_element_type=F32) + bs_ref[...]
        od_ref[...] = jnp.dot(xv, wd_ref[...], preferred_element_type=F32) + bd_ref[...]

    return pl.pallas_call(
        body,
        out_shape=(jax.ShapeDtypeStruct((N, H), F32),
                   jax.ShapeDtypeStruct((N, H), F32)),
    )(x, W_src, b_src.reshape(1, -1), W_dst, b_dst.reshape(1, -1))


def _edge_mlp(edge_attr, s, W_ea1, b_ea1, W_ea2, b_ea2, W_te1, b_te1, W_te2, b_te2):
    E, DE = edge_attr.shape
    H = W_ea1.shape[1]
    D = W_te2.shape[1]
    K = 2560
    assert E % K == 0
    grid = E // K

    def body(ea_ref, s_ref, w1, c1, w2, c2, w3, c3, w4, c4, out_ref):
        ec = jnp.dot(jax.nn.relu(jnp.dot(ea_ref[...], w1[...], preferred_element_type=F32) + c1[...]),
                     w2[...], preferred_element_type=F32) + c2[...]
        t = jax.nn.relu(ec + s_ref[...])
        u = jax.nn.relu(jnp.dot(t, w3[...], preferred_element_type=F32) + c3[...])
        out_ref[...] = jnp.dot(u, w4[...], preferred_element_type=F32) + c4[...]

    wspec = lambda r, c: pl.BlockSpec((r, c), lambda i: (0, 0))
    return pl.pallas_call(
        body,
        grid=(grid,),
        in_specs=[
            pl.BlockSpec((K, DE), lambda i: (i, 0)),
            pl.BlockSpec((K, H), lambda i: (i, 0)),
            wspec(DE, H), wspec(1, H), wspec(H, H), wspec(1, H),
            wspec(H, H), wspec(1, H), wspec(H, D), wspec(1, D),
        ],
        out_specs=pl.BlockSpec((K, D), lambda i: (i, 0)),
        out_shape=jax.ShapeDtypeStruct((E, D), F32),
    )(edge_attr, s,
      W_ea1, b_ea1.reshape(1, -1), W_ea2, b_ea2.reshape(1, -1),
      W_te1, b_te1.reshape(1, -1), W_te2, b_te2.reshape(1, -1))


def _final_mlp(x, agg_parts, W_phid, b_phid, W_phie, b_phie, W_phi, b_phi):
    N, D = x.shape
    H = W_phid.shape[1]

    def body(x_ref, a_ref, wd, bd, we, be, wp, bp, out_ref):
        agg = a_ref[0] + a_ref[1]
        h = jax.nn.relu(jnp.dot(x_ref[...], wd[...], preferred_element_type=F32)
                        + jnp.dot(agg, we[...], preferred_element_type=F32)
                        + bd[...] + be[...])
        out_ref[...] = jnp.dot(h, wp[...], preferred_element_type=F32) + bp[...]

    return pl.pallas_call(
        body,
        out_shape=jax.ShapeDtypeStruct((N, D), F32),
    )(x, agg_parts, W_phid, b_phid.reshape(1, -1),
      W_phie, b_phie.reshape(1, -1), W_phi, b_phi.reshape(1, -1))


_C = 128


def _sc_gather_sum(xw_src, xw_dst, src, dst, NC, NS):
    N, H = xw_src.shape
    E = src.shape[0]
    assert E % _C == 0
    TOT = E // _C
    NW = NC * NS
    KMAX = (TOT + NW - 1) // NW
    mesh = plsc.VectorSubcoreMesh(core_axis_name="c", subcore_axis_name="s")

    PAIRS = (KMAX + 1) // 2

    @functools.partial(
        pl.kernel, mesh=mesh,
        out_type=jax.ShapeDtypeStruct((E, H), F32),
        scratch_types=[
            pltpu.VMEM((2, _C), jnp.int32), pltpu.VMEM((2, _C), jnp.int32),
            pltpu.VMEM((2, _C, H), F32), pltpu.VMEM((2, _C, H), F32),
            pltpu.SemaphoreType.DMA, pltpu.SemaphoreType.DMA,
            pltpu.SemaphoreType.DMA, pltpu.SemaphoreType.DMA,
        ],
    )
    def k(xs_hbm, xd_hbm, src_hbm, dst_hbm, out_hbm,
          sidx, didx, gs, gd, si0, si1, sg0, sg1):
        wid = lax.axis_index("s") * NC + lax.axis_index("c")
        sem_i = (si0, si1)
        sem_g = (sg0, sg1)

        def fire_idx(chunk, slot):
            base = chunk * _C
            pltpu.async_copy(src_hbm.at[pl.ds(base, _C)], sidx.at[slot], sem_i[slot])
            pltpu.async_copy(dst_hbm.at[pl.ds(base, _C)], didx.at[slot], sem_i[slot])

        def wait_idx(slot):
            pltpu.make_async_copy(src_hbm.at[pl.ds(0, _C)], sidx.at[slot], sem_i[slot]).wait()
            pltpu.make_async_copy(dst_hbm.at[pl.ds(0, _C)], didx.at[slot], sem_i[slot]).wait()

        def fire_gather(slot):
            pltpu.async_copy(xs_hbm.at[sidx.at[slot]], gs.at[slot], sem_g[slot])
            pltpu.async_copy(xd_hbm.at[didx.at[slot]], gd.at[slot], sem_g[slot])

        def wait_gather(slot):
            pltpu.make_async_copy(xs_hbm.at[sidx.at[slot]], gs.at[slot], sem_g[slot]).wait()
            pltpu.make_async_copy(xd_hbm.at[didx.at[slot]], gd.at[slot], sem_g[slot]).wait()

        def process(chunk, slot):
            wait_gather(slot)

            def row(r, c):
                for j in range(H // 16):
                    sl = pl.ds(j * 16, 16)
                    gs[slot, r, sl] = gs[slot, r, sl] + gd[slot, r, sl]
                return c

            lax.fori_loop(0, _C, row, 0)
            pltpu.sync_copy(gs.at[slot], out_hbm.at[pl.ds(chunk * _C, _C)])

        c0 = wid

        @pl.when(c0 < TOT)
        def _():
            fire_idx(c0, 0)
            wait_idx(0)
            fire_gather(0)

        @pl.when(c0 + NW < TOT)
        def _():
            fire_idx(c0 + NW, 1)

        def pair(kk, carry):
            ca = (2 * kk) * NW + wid
            cb = ca + NW
            cc = cb + NW
            cd = cc + NW

            @pl.when(cb < TOT)
            def _():
                wait_idx(1)
                fire_gather(1)

            @pl.when(ca < TOT)
            def _():
                process(ca, 0)

            @pl.when(cc < TOT)
            def _():
                fire_idx(cc, 0)
                wait_idx(0)
                fire_gather(0)

            @pl.when(cb < TOT)
            def _():
                process(cb, 1)

            @pl.when(cd < TOT)
            def _():
                fire_idx(cd, 1)

            return carry

        lax.fori_loop(0, PAIRS, pair, 0)

    return k(xw_src, xw_dst, src, dst)


def _sc_scatter_agg(x, e_emb, src, dst, NC, NS):
    CD = 64
    N, H = x.shape
    E = src.shape[0]
    assert E % CD == 0
    TOT = E // CD
    per_core = TOT // NC
    KMAX = (per_core + NS - 1) // NS
    RPT = ((N + NS * CD - 1) // (NS * CD)) * CD
    NPAD = NS * RPT
    mesh = plsc.VectorSubcoreMesh(core_axis_name="c", subcore_axis_name="s")

    PAIRS = (KMAX + 1) // 2

    @functools.partial(
        pl.kernel, mesh=mesh,
        out_type=jax.ShapeDtypeStruct((NC, NPAD, H), F32),
        scratch_types=[
            pltpu.VMEM((2, CD), jnp.int32), pltpu.VMEM((2, CD), jnp.int32),
            pltpu.VMEM((2, CD, H), F32), pltpu.VMEM((2, CD, H), F32),
            pltpu.VMEM_SHARED((NPAD, H), F32),
            pltpu.SemaphoreType.DMA, pltpu.SemaphoreType.DMA,
            pltpu.SemaphoreType.DMA, pltpu.SemaphoreType.DMA,
        ],
    )
    def k(x_hbm, ee_hbm, src_hbm, dst_hbm, out_hbm,
          sidx, didx, gx, em, acc, si0, si1, sg0, sg1):
        cid = lax.axis_index("c")
        sid = lax.axis_index("s")
        sem_i = (si0, si1)
        sem_g = (sg0, sg1)
        lim = (cid + 1) * per_core

        def zrow(r, c):
            for j in range(H // 16):
                em[0, r, pl.ds(j * 16, 16)] = jnp.zeros((16,), F32)
            return c

        lax.fori_loop(0, CD, zrow, 0)
        for j in range(RPT // CD):
            pltpu.sync_copy(em.at[0], acc.at[pl.ds(sid * RPT + j * CD, CD)])
        plsc.subcore_barrier()

        def fire_idx(chunk, slot):
            base = chunk * CD
            pltpu.async_copy(src_hbm.at[pl.ds(base, CD)], sidx.at[slot], sem_i[slot])
            pltpu.async_copy(dst_hbm.at[pl.ds(base, CD)], didx.at[slot], sem_i[slot])
            pltpu.async_copy(ee_hbm.at[pl.ds(base, CD)], em.at[slot], sem_i[slot])

        def wait_idx(slot):
            pltpu.make_async_copy(src_hbm.at[pl.ds(0, CD)], sidx.at[slot], sem_i[slot]).wait()
            pltpu.make_async_copy(dst_hbm.at[pl.ds(0, CD)], didx.at[slot], sem_i[slot]).wait()
            pltpu.make_async_copy(ee_hbm.at[pl.ds(0, CD)], em.at[slot], sem_i[slot]).wait()

        def fire_gather(slot):
            pltpu.async_copy(x_hbm.at[sidx.at[slot]], gx.at[slot], sem_g[slot])

        def process(slot):
            pltpu.make_async_copy(x_hbm.at[sidx.at[slot]], gx.at[slot], sem_g[slot]).wait()

            def row(r, c):
                for j in range(H // 16):
                    sl = pl.ds(j * 16, 16)
                    em[slot, r, sl] = em[slot, r, sl] * gx[slot, r, sl]
                return c

            lax.fori_loop(0, CD, row, 0)
            pltpu.sync_copy(em.at[slot], acc.at[didx.at[slot]], add=True)

        c0 = cid * per_core + sid

        @pl.when(c0 < lim)
        def _():
            fire_idx(c0, 0)
            wait_idx(0)
            fire_gather(0)

        @pl.when(c0 + NS < lim)
        def _():
            fire_idx(c0 + NS, 1)

        def pair(kk, carry):
            ca = cid * per_core + (2 * kk) * NS + sid
            cb = ca + NS
            cc = cb + NS
            cd = cc + NS

            @pl.when(cb < lim)
            def _():
                wait_idx(1)
                fire_gather(1)

            @pl.when(ca < lim)
            def _():
                process(0)

            @pl.when(cc < lim)
            def _():
                fire_idx(cc, 0)
                wait_idx(0)
                fire_gather(0)

            @pl.when(cb < lim)
            def _():
                process(1)

            @pl.when(cd < lim)
            def _():
                fire_idx(cd, 1)

            return carry

        lax.fori_loop(0, PAIRS, pair, 0)
        plsc.subcore_barrier()
        for j in range(RPT // CD):
            off = sid * RPT + j * CD
            pltpu.sync_copy(acc.at[pl.ds(off, CD)],
                            out_hbm.at[cid, pl.ds(off, CD)])

    return k(x, e_emb, src, dst)[:, :N, :]


def kernel(node_feat, edge_attr, W_ea1, b_ea1, W_ea2, b_ea2, W_src, b_src,
           W_dst, b_dst, W_te1, b_te1, W_te2, b_te2, W_phid, b_phid,
           W_phie, b_phie, W_phi, b_phi, edge_index):
    info = plsc.get_sparse_core_info()
    NC, NS = info.num_cores, info.num_subcores
    src = edge_index[0]
    dst = edge_index[1]
    xw_src, xw_dst = _node_proj(node_feat, W_src, b_src, W_dst, b_dst)
    s = _sc_gather_sum(xw_src, xw_dst, src, dst, NC, NS)
    e_emb = _edge_mlp(edge_attr, s, W_ea1, b_ea1, W_ea2, b_ea2,
                      W_te1, b_te1, W_te2, b_te2)
    agg_parts = _sc_scatter_agg(node_feat, e_emb, src, dst, NC, NS)
    return _final_mlp(node_feat, agg_parts, W_phid, b_phid,
                      W_phie, b_phie, W_phi, b_phi)


# speedup vs baseline: 4.2446x; 1.2379x over previous
import functools

import jax
import jax.numpy as jnp
from jax import lax
from jax.experimental import pallas as pl
from jax.experimental.pallas import tpu as pltpu
from jax.experimental.pallas import tpu_sc as plsc

F32 = jnp.float32


def _node_proj(x, W_src, b_src, W_dst, b_dst):
    N, D = x.shape
    H = W_src.shape[1]

    def body(x_ref, ws_ref, bs_ref, wd_ref, bd_ref, os_ref, od_ref):
        xv = x_ref[...]
        os_ref[...] = jnp.dot(xv, ws_ref[...], preferred_element_type=F32) + bs_ref[...]
        od_ref[...] = jnp.dot(xv, wd_ref[...], preferred_element_type=F32) + bd_ref[...]

    return pl.pallas_call(
        body,
        out_shape=(jax.ShapeDtypeStruct((N, H), F32),
                   jax.ShapeDtypeStruct((N, H), F32)),
    )(x, W_src, b_src.reshape(1, -1), W_dst, b_dst.reshape(1, -1))


def _edge_mlp(edge_attr, s, W_ea1, b_ea1, W_ea2, b_ea2, W_te1, b_te1, W_te2, b_te2):
    E, DE = edge_attr.shape
    H = W_ea1.shape[1]
    D = W_te2.shape[1]
    K = 4000
    assert E % K == 0
    grid = E // K

    def body(ea_ref, s_ref, w1, c1, w2, c2, w3, c3, w4, c4, out_ref):
        ec = jnp.dot(jax.nn.relu(jnp.dot(ea_ref[...], w1[...], preferred_element_type=F32) + c1[...]),
                     w2[...], preferred_element_type=F32) + c2[...]
        t = jax.nn.relu(ec + s_ref[...])
        u = jax.nn.relu(jnp.dot(t, w3[...], preferred_element_type=F32) + c3[...])
        out_ref[...] = jnp.dot(u, w4[...], preferred_element_type=F32) + c4[...]

    wspec = lambda r, c: pl.BlockSpec((r, c), lambda i: (0, 0))
    return pl.pallas_call(
        body,
        grid=(grid,),
        in_specs=[
            pl.BlockSpec((K, DE), lambda i: (i, 0)),
            pl.BlockSpec((K, H), lambda i: (i, 0)),
            wspec(DE, H), wspec(1, H), wspec(H, H), wspec(1, H),
            wspec(H, H), wspec(1, H), wspec(H, D), wspec(1, D),
        ],
        out_specs=pl.BlockSpec((K, D), lambda i: (i, 0)),
        out_shape=jax.ShapeDtypeStruct((E, D), F32),
    )(edge_attr, s,
      W_ea1, b_ea1.reshape(1, -1), W_ea2, b_ea2.reshape(1, -1),
      W_te1, b_te1.reshape(1, -1), W_te2, b_te2.reshape(1, -1))


def _final_mlp(x, agg_parts, W_phid, b_phid, W_phie, b_phie, W_phi, b_phi):
    N, D = x.shape
    H = W_phid.shape[1]

    def body(x_ref, a_ref, wd, bd, we, be, wp, bp, out_ref):
        agg = a_ref[0] + a_ref[1]
        h = jax.nn.relu(jnp.dot(x_ref[...], wd[...], preferred_element_type=F32)
                        + jnp.dot(agg, we[...], preferred_element_type=F32)
                        + bd[...] + be[...])
        out_ref[...] = jnp.dot(h, wp[...], preferred_element_type=F32) + bp[...]

    return pl.pallas_call(
        body,
        out_shape=jax.ShapeDtypeStruct((N, D), F32),
    )(x, agg_parts, W_phid, b_phid.reshape(1, -1),
      W_phie, b_phie.reshape(1, -1), W_phi, b_phi.reshape(1, -1))


_C = 128


def _sc_gather_sum(xw_src, xw_dst, src, dst, NC, NS):
    N, H = xw_src.shape
    E = src.shape[0]
    S = 3
    assert E % _C == 0
    TOT = E // _C
    NW = NC * NS
    KMAX = (TOT + NW - 1) // NW
    NTRIP = (KMAX + S - 1) // S
    mesh = plsc.VectorSubcoreMesh(core_axis_name="c", subcore_axis_name="s")

    @functools.partial(
        pl.kernel, mesh=mesh,
        out_type=jax.ShapeDtypeStruct((E, H), F32),
        scratch_types=[
            pltpu.VMEM((S, _C), jnp.int32), pltpu.VMEM((S, _C), jnp.int32),
            pltpu.VMEM((S, _C, H), F32), pltpu.VMEM((S, _C, H), F32),
        ] + [pltpu.SemaphoreType.DMA] * (3 * S),
    )
    def k(xs_hbm, xd_hbm, src_hbm, dst_hbm, out_hbm,
          sidx, didx, gs, gd, *sems):
        wid = lax.axis_index("s") * NC + lax.axis_index("c")
        sem_i = sems[0:S]
        sem_g = sems[S:2 * S]
        sem_t = sems[2 * S:3 * S]

        def chunk(ph):
            return ph * NW + wid

        def fire_idx(c, slot):
            base = c * _C
            pltpu.async_copy(src_hbm.at[pl.ds(base, _C)], sidx.at[slot], sem_i[slot])
            pltpu.async_copy(dst_hbm.at[pl.ds(base, _C)], didx.at[slot], sem_i[slot])

        def wait_idx(slot):
            pltpu.make_async_copy(src_hbm.at[pl.ds(0, _C)], sidx.at[slot], sem_i[slot]).wait()
            pltpu.make_async_copy(dst_hbm.at[pl.ds(0, _C)], didx.at[slot], sem_i[slot]).wait()

        def fire_gather(slot):
            pltpu.async_copy(xs_hbm.at[sidx.at[slot]], gs.at[slot], sem_g[slot])
            pltpu.async_copy(xd_hbm.at[didx.at[slot]], gd.at[slot], sem_g[slot])

        def wait_gather(slot):
            pltpu.make_async_copy(xs_hbm.at[sidx.at[slot]], gs.at[slot], sem_g[slot]).wait()
            pltpu.make_async_copy(xd_hbm.at[didx.at[slot]], gd.at[slot], sem_g[slot]).wait()

        def wait_store(slot):
            pltpu.make_async_copy(gs.at[slot], out_hbm.at[pl.ds(0, _C)], sem_t[slot]).wait()

        def process(c, slot):
            wait_gather(slot)

            def row(r, cr):
                for j in range(H // 16):
                    sl = pl.ds(j * 16, 16)
                    gs[slot, r, sl] = gs[slot, r, sl] + gd[slot, r, sl]
                return cr

            lax.fori_loop(0, _C, row, 0)
            pltpu.async_copy(gs.at[slot], out_hbm.at[pl.ds(c * _C, _C)], sem_t[slot])

        @pl.when(chunk(0) < TOT)
        def _():
            fire_idx(chunk(0), 0)

        @pl.when(chunk(1) < TOT)
        def _():
            fire_idx(chunk(1), 1)

        @pl.when(chunk(0) < TOT)
        def _():
            wait_idx(0)
            fire_gather(0)

        def trip(it, carry):
            for j in range(S):
                ph = it * S + j
                cur, nxt, nn = j, (j + 1) % S, (j + 2) % S
                c, c1, c2 = chunk(ph), chunk(ph + 1), chunk(ph + 2)

                @pl.when(c2 < TOT)
                def _():
                    fire_idx(c2, nn)

                @pl.when(c1 < TOT)
                def _():
                    wait_idx(nxt)

                    @pl.when(ph + 1 >= S)
                    def _():
                        wait_store(nxt)

                    fire_gather(nxt)

                @pl.when(c < TOT)
                def _():
                    process(c, cur)

            return carry

        lax.fori_loop(0, NTRIP, trip, 0)

        for j in range(S):
            @pl.when(chunk(S - 1) < TOT)
            def _():
                wait_store(j)

    return k(xw_src, xw_dst, src, dst)


def _sc_scatter_agg(x, e_emb, src, dst, NC, NS):
    CD = 64
    N, H = x.shape
    E = src.shape[0]
    S = 3
    assert E % CD == 0
    TOT = E // CD
    per_core = TOT // NC
    KMAX = (per_core + NS - 1) // NS
    NTRIP = (KMAX + S - 1) // S
    RPT = ((N + 8 * NS - 1) // (8 * NS)) * 8
    NPAD = NS * RPT
    mesh = plsc.VectorSubcoreMesh(core_axis_name="c", subcore_axis_name="s")

    @functools.partial(
        pl.kernel, mesh=mesh,
        out_type=jax.ShapeDtypeStruct((NC, NPAD, H), F32),
        scratch_types=[
            pltpu.VMEM((S, CD), jnp.int32), pltpu.VMEM((S, CD), jnp.int32),
            pltpu.VMEM((S, CD, H), F32), pltpu.VMEM((S, CD, H), F32),
            pltpu.VMEM_SHARED((NPAD, H), F32),
        ] + [pltpu.SemaphoreType.DMA] * (3 * S),
    )
    def k(x_hbm, ee_hbm, src_hbm, dst_hbm, out_hbm,
          sidx, didx, gx, em, acc, *sems):
        cid = lax.axis_index("c")
        sid = lax.axis_index("s")
        sem_i = sems[0:S]
        sem_g = sems[S:2 * S]
        sem_t = sems[2 * S:3 * S]
        lim = (cid + 1) * per_core

        def chunk(ph):
            return cid * per_core + ph * NS + sid

        def zrow(r, c):
            for j in range(H // 16):
                em[0, r, pl.ds(j * 16, 16)] = jnp.zeros((16,), F32)
            return c

        lax.fori_loop(0, CD, zrow, 0)
        full, rem = RPT // CD, RPT % CD
        for j in range(full):
            pltpu.sync_copy(em.at[0], acc.at[pl.ds(sid * RPT + j * CD, CD)])
        if rem:
            pltpu.sync_copy(em.at[0, pl.ds(0, rem)],
                            acc.at[pl.ds(sid * RPT + full * CD, rem)])
        plsc.subcore_barrier()

        def fire_idx(c, slot):
            base = c * CD
            pltpu.async_copy(src_hbm.at[pl.ds(base, CD)], sidx.at[slot], sem_i[slot])
            pltpu.async_copy(dst_hbm.at[pl.ds(base, CD)], didx.at[slot], sem_i[slot])

        def wait_idx(slot):
            pltpu.make_async_copy(src_hbm.at[pl.ds(0, CD)], sidx.at[slot], sem_i[slot]).wait()
            pltpu.make_async_copy(dst_hbm.at[pl.ds(0, CD)], didx.at[slot], sem_i[slot]).wait()

        def fire_gather(c, slot):
            pltpu.async_copy(x_hbm.at[sidx.at[slot]], gx.at[slot], sem_g[slot])
            pltpu.async_copy(ee_hbm.at[pl.ds(c * CD, CD)], em.at[slot], sem_g[slot])

        def wait_gather(slot):
            pltpu.make_async_copy(x_hbm.at[sidx.at[slot]], gx.at[slot], sem_g[slot]).wait()
            pltpu.make_async_copy(ee_hbm.at[pl.ds(0, CD)], em.at[slot], sem_g[slot]).wait()

        def wait_scat(slot):
            pltpu.make_async_copy(em.at[slot], acc.at[pl.ds(0, CD)], sem_t[slot]).wait()

        def process(slot):
            wait_gather(slot)

            def row(r, cr):
                for j in range(H // 16):
                    sl = pl.ds(j * 16, 16)
                    em[slot, r, sl] = em[slot, r, sl] * gx[slot, r, sl]
                return cr

            lax.fori_loop(0, CD, row, 0)
            pltpu.async_copy(em.at[slot], acc.at[didx.at[slot]], sem_t[slot], add=True)

        @pl.when(chunk(0) < lim)
        def _():
            fire_idx(chunk(0), 0)

        @pl.when(chunk(1) < lim)
        def _():
            fire_idx(chunk(1), 1)

        @pl.when(chunk(0) < lim)
        def _():
            wait_idx(0)
            fire_gather(chunk(0), 0)

        def trip(it, carry):
            for j in range(S):
                ph = it * S + j
                cur, nxt, nn = j, (j + 1) % S, (j + 2) % S
                c, c1, c2 = chunk(ph), chunk(ph + 1), chunk(ph + 2)

                @pl.when(c2 < lim)
                def _():
                    @pl.when(ph >= 1)
                    def _():
                        wait_scat(nn)

                    fire_idx(c2, nn)

                @pl.when(c1 < lim)
                def _():
                    wait_idx(nxt)
                    fire_gather(c1, nxt)

                @pl.when(c < lim)
                def _():
                    process(cur)

            return carry

        lax.fori_loop(0, NTRIP, trip, 0)

        for j in range(S):
            @pl.when(chunk(S - 1) < lim)
            def _():
                wait_scat(j)

        plsc.subcore_barrier()
        pltpu.sync_copy(acc.at[pl.ds(sid * RPT, RPT)],
                        out_hbm.at[cid, pl.ds(sid * RPT, RPT)])

    return k(x, e_emb, src, dst)[:, :N, :]


def kernel(node_feat, edge_attr, W_ea1, b_ea1, W_ea2, b_ea2, W_src, b_src,
           W_dst, b_dst, W_te1, b_te1, W_te2, b_te2, W_phid, b_phid,
           W_phie, b_phie, W_phi, b_phi, edge_index):
    info = plsc.get_sparse_core_info()
    NC, NS = info.num_cores, info.num_subcores
    src = edge_index[0]
    dst = edge_index[1]
    xw_src, xw_dst = _node_proj(node_feat, W_src, b_src, W_dst, b_dst)
    s = _sc_gather_sum(xw_src, xw_dst, src, dst, NC, NS)
    e_emb = _edge_mlp(edge_attr, s, W_ea1, b_ea1, W_ea2, b_ea2,
                      W_te1, b_te1, W_te2, b_te2)
    agg_parts = _sc_scatter_agg(node_feat, e_emb, src, dst, NC, NS)
    return _final_mlp(node_feat, agg_parts, W_phid, b_phid,
                      W_phie, b_phie, W_phi, b_phi)
